# baseline (speedup 1.0000x reference)
import jax
import jax.numpy as jnp
from jax import lax
from jax.experimental import pallas as pl
from jax.experimental.pallas import tpu as pltpu
from jax.experimental.pallas import tpu_sc as plsc

N = 10000
D = 128
E = 320000

NC = 2
NS = 16
NW = NC * NS
L = 16

C = 88
CPW0 = 216
CPW1 = 24
NBUF = 4
LA = NBUF - 1
NSLOT = 6
PER = 12
ATOT = NS * (CPW0 + CPW1)
AEPAD = ATOT * C
RPT = 640
NPAD = NS * RPT

RB = 400
GRID = N // RB

_mesh = plsc.VectorSubcoreMesh(
    core_axis_name="c", subcore_axis_name="s", num_cores=NC, num_subcores=NS
)


def _deg_body(dst_hbm, cnt_hbm, dst_v, ones_v, zeros_v, cnt_s):
    cid = lax.axis_index("c")
    sid = lax.axis_index("s")
    wid = cid * NS + sid
    cpw = lax.select(cid == 0, CPW0, CPW1)
    for i in range(-(-C // L)):
        ones_v[pl.ds(i * L, L)] = jnp.full((L,), 1.0, jnp.float32)
    for i in range(RPT // L):
        zeros_v[pl.ds(i * L, L)] = jnp.zeros((L,), jnp.float32)
    pltpu.sync_copy(dst_hbm.at[wid], dst_v)
    pltpu.sync_copy(zeros_v, cnt_s.at[pl.ds(sid * RPT, RPT)])
    plsc.subcore_barrier()

    @pl.loop(0, cpw)
    def _(j):
        pltpu.sync_copy(ones_v.at[pl.ds(0, C)], cnt_s.at[dst_v.at[j]],
                        add=True)

    plsc.subcore_barrier()
    pltpu.sync_copy(cnt_s.at[pl.ds(sid * RPT, RPT)],
                    cnt_hbm.at[cid, pl.ds(sid * RPT, RPT)])


_deg_call = pl.kernel(
    _deg_body,
    out_type=jax.ShapeDtypeStruct((NC, NPAD), jnp.float32),
    mesh=_mesh,
    scratch_types=[
        pltpu.VMEM((CPW0, C), jnp.int32),
        pltpu.VMEM((((C + L - 1) // L) * L,), jnp.float32),
        pltpu.VMEM((RPT,), jnp.float32),
        pltpu.VMEM_SHARED((NPAD,), jnp.float32),
    ],
)


def _agg_body(y_hbm, src_hbm, dst_hbm, out_hbm, src_ring, dst_ring, rows_v,
              *sems):
    gsems = sems[0:NBUF]
    ssems = sems[NBUF:2 * NBUF]
    isems = sems[2 * NBUF:2 * NBUF + NSLOT]
    idems = sems[2 * NBUF + NSLOT:2 * NBUF + 2 * NSLOT]
    acc_s = sems[-1]
    cid = lax.axis_index("c")
    sid = lax.axis_index("s")
    wid = cid * NS + sid
    cpw = lax.select(cid == 0, CPW0, CPW1)

    def isdesc(j, s):
        return pltpu.make_async_copy(src_hbm.at[wid, j], src_ring.at[s],
                                     isems[s])

    def iddesc(j, s):
        return pltpu.make_async_copy(dst_hbm.at[wid, j], dst_ring.at[s],
                                     idems[s])

    def gdesc(s, b):
        return pltpu.make_async_copy(y_hbm.at[src_ring.at[s]], rows_v.at[b],
                                     gsems[b])

    def sdesc(s, b):
        return pltpu.make_async_copy(rows_v.at[b], acc_s.at[dst_ring.at[s]],
                                     ssems[b])

    for k in range(NSLOT - 1):
        isdesc(k, k).start()
        iddesc(k, k).start()
    ZR = 40
    for r in range(ZR):
        for i in range(D // L):
            rows_v[0, r, pl.ds(i * L, L)] = jnp.zeros((L,), jnp.float32)
    zdescs = [
        pltpu.make_async_copy(rows_v.at[0, pl.ds(0, ZR)],
                              acc_s.at[pl.ds(sid * RPT + k * ZR, ZR)],
                              ssems[0])
        for k in range(RPT // ZR)
    ]
    for zd in zdescs:
        zd.start()
    for zd in zdescs:
        zd.wait()
    plsc.subcore_barrier()
    for k in range(LA):
        isdesc(k, k).wait()
        gdesc(k, k).start()

    def chunk(j, r):
        b, s = r % NBUF, r % NSLOT
        gdesc(s, b).wait()
        iddesc(j, s).wait()
        sdesc(s, b).start(add=True)

        @pl.when(j >= 1)
        def _():
            sdesc((s + NSLOT - 1) % NSLOT, (b + NBUF - 1) % NBUF).wait()

        @pl.when(j + NSLOT - 1 < cpw)
        def _():
            isdesc(j + NSLOT - 1, (s + NSLOT - 1) % NSLOT).start()
            iddesc(j + NSLOT - 1, (s + NSLOT - 1) % NSLOT).start()

        @pl.when(j + LA < cpw)
        def _():
            isdesc(j + LA, (s + LA) % NSLOT).wait()
            gdesc((s + LA) % NSLOT, (b + LA) % NBUF).start()

    @pl.loop(0, cpw // PER)
    def _(g):
        for k in range(PER):
            chunk(g * PER + k, k)

    sdesc((PER - 1) % NSLOT, (PER - 1) % NBUF).wait()
    plsc.subcore_barrier()
    pltpu.sync_copy(acc_s.at[pl.ds(sid * RPT, RPT)],
                    out_hbm.at[cid, pl.ds(sid * RPT, RPT)])


_agg_call = pl.kernel(
    _agg_body,
    out_type=jax.ShapeDtypeStruct((NC, NPAD, D), jnp.float32),
    mesh=_mesh,
    scratch_types=[
        pltpu.VMEM((NSLOT, C), jnp.int32),
        pltpu.VMEM((NSLOT, C), jnp.int32),
        pltpu.VMEM((NBUF, C, D), jnp.float32),
    ] + [pltpu.SemaphoreType.DMA] * (2 * NBUF + 2 * NSLOT) + [
        pltpu.VMEM_SHARED((NPAD, D), jnp.float32),
    ],
)


def _dinv_block(cnt_blk):
    return lax.rsqrt(1.0 + cnt_blk[:, 0] + cnt_blk[:, 1])[:, None]


def _lin_body(x_ref, w_ref, cnt_ref, o_ref):
    o_ref[...] = (
        jnp.dot(x_ref[...], w_ref[...], preferred_element_type=jnp.float32)
        * _dinv_block(cnt_ref[...])
    )


def _mid_body(g_ref, y_ref, cnt_ref, b_ref, w_ref, o_ref):
    g = g_ref[...]
    dinv = _dinv_block(cnt_ref[...])
    h = jnp.maximum(dinv * (g[0] + g[1] + y_ref[...]) + b_ref[...], 0.0)
    o_ref[...] = (
        jnp.dot(h, w_ref[...], preferred_element_type=jnp.float32) * dinv
    )


def _fin_body(g_ref, y_ref, cnt_ref, b_ref, o_ref):
    g = g_ref[...]
    dinv = _dinv_block(cnt_ref[...])
    o_ref[...] = dinv * (g[0] + g[1] + y_ref[...]) + b_ref[...]


_row_spec = pl.BlockSpec((RB, D), lambda i: (i, 0))
_cnt_spec = pl.BlockSpec((RB, NC), lambda i: (i, 0))
_g_spec = pl.BlockSpec((NC, RB, D), lambda i: (0, i, 0))
_w_spec = pl.BlockSpec((D, D), lambda i: (0, 0))
_b_spec = pl.BlockSpec((1, D), lambda i: (0, 0))
_out_shape = jax.ShapeDtypeStruct((NPAD, D), jnp.float32)

_lin_call = pl.pallas_call(
    _lin_body, grid=(GRID,),
    in_specs=[_row_spec, _w_spec, _cnt_spec],
    out_specs=_row_spec, out_shape=_out_shape,
)

_mid_call = pl.pallas_call(
    _mid_body, grid=(GRID,),
    in_specs=[_g_spec, _row_spec, _cnt_spec, _b_spec, _w_spec],
    out_specs=_row_spec, out_shape=_out_shape,
)

_fin_call = pl.pallas_call(
    _fin_body, grid=(GRID,),
    in_specs=[_g_spec, _row_spec, _cnt_spec, _b_spec],
    out_specs=_row_spec, out_shape=jax.ShapeDtypeStruct((N, D), jnp.float32),
)


def kernel(x, edge_index, W1, b1, W2, b2):
    src = edge_index[0].astype(jnp.int32)
    dst = edge_index[1].astype(jnp.int32)
    apad = jnp.full((AEPAD - E,), N, jnp.int32)

    def _split3(flat):
        a0 = flat[:NS * CPW0 * C].reshape(NS, CPW0, C)
        a1 = flat[NS * CPW0 * C:].reshape(NS, CPW1, C)
        a1 = jnp.pad(a1, ((0, 0), (0, CPW0 - CPW1), (0, 0)),
                     constant_values=N)
        return jnp.concatenate([a0, a1], axis=0)

    srca = _split3(jnp.concatenate([src, apad]))
    dsta = _split3(jnp.concatenate([dst, apad]))
    b1r = b1.reshape(1, D)
    b2r = b2.reshape(1, D)

    cnt = _deg_call(dsta).T
    y1 = _lin_call(x, W1, cnt)
    g1 = _agg_call(y1, srca, dsta)
    y2 = _mid_call(g1, y1, cnt, b1r, W2)
    g2 = _agg_call(y2, srca, dsta)
    return _fin_call(g2, y2, cnt, b2r)


# speedup vs baseline: 3.4744x; 3.4744x over previous
import jax
import jax.numpy as jnp
from jax import lax
from jax.experimental import pallas as pl
from jax.experimental.pallas import tpu as pltpu
from jax.experimental.pallas import tpu_sc as plsc

N = 10000
D = 128
E = 320000

NC = 2
NS = 16
NW = NC * NS
L = 16

C = 88
CPW0 = 216
CPW1 = 12
NBUF = 4
LA = NBUF - 1
NSLOT = 6
PER = 12
ATOT = NS * (CPW0 + CPW1)
AEPAD = ATOT * C
RPT = 640
NPAD = NS * RPT

RB = 400
GRID = N // RB

_mesh = plsc.VectorSubcoreMesh(
    core_axis_name="c", subcore_axis_name="s", num_cores=NC, num_subcores=NS
)


def _deg_body(dst_hbm, cnt_hbm, dst_v, ones_v, zeros_v, cnt_s):
    cid = lax.axis_index("c")
    sid = lax.axis_index("s")
    wid = cid * NS + sid
    cpw = lax.select(cid == 0, CPW0, CPW1)
    for i in range(-(-C // L)):
        ones_v[pl.ds(i * L, L)] = jnp.full((L,), 1.0, jnp.float32)
    for i in range(RPT // L):
        zeros_v[pl.ds(i * L, L)] = jnp.zeros((L,), jnp.float32)
    pltpu.sync_copy(dst_hbm.at[wid], dst_v)
    pltpu.sync_copy(zeros_v, cnt_s.at[pl.ds(sid * RPT, RPT)])
    plsc.subcore_barrier()

    @pl.loop(0, cpw)
    def _(j):
        pltpu.sync_copy(ones_v.at[pl.ds(0, C)], cnt_s.at[dst_v.at[j]],
                        add=True)

    plsc.subcore_barrier()
    pltpu.sync_copy(cnt_s.at[pl.ds(sid * RPT, RPT)],
                    cnt_hbm.at[cid, pl.ds(sid * RPT, RPT)])


_deg_call = pl.kernel(
    _deg_body,
    out_type=jax.ShapeDtypeStruct((NC, NPAD), jnp.float32),
    mesh=_mesh,
    scratch_types=[
        pltpu.VMEM((CPW0, C), jnp.int32),
        pltpu.VMEM((((C + L - 1) // L) * L,), jnp.float32),
        pltpu.VMEM((RPT,), jnp.float32),
        pltpu.VMEM_SHARED((NPAD,), jnp.float32),
    ],
)


def _agg_body(y_hbm, src_hbm, dst_hbm, out_hbm, src_ring, dst_ring, rows_v,
              *sems):
    gsems = sems[0:NBUF]
    ssems = sems[NBUF:2 * NBUF]
    isems = sems[2 * NBUF:2 * NBUF + NSLOT]
    idems = sems[2 * NBUF + NSLOT:2 * NBUF + 2 * NSLOT]
    acc_s = sems[-1]
    cid = lax.axis_index("c")
    sid = lax.axis_index("s")
    wid = cid * NS + sid
    cpw = lax.select(cid == 0, CPW0, CPW1)

    def isdesc(j, s):
        return pltpu.make_async_copy(src_hbm.at[wid, j], src_ring.at[s],
                                     isems[s])

    def iddesc(j, s):
        return pltpu.make_async_copy(dst_hbm.at[wid, j], dst_ring.at[s],
                                     idems[s])

    def gdesc(s, b):
        return pltpu.make_async_copy(y_hbm.at[src_ring.at[s]], rows_v.at[b],
                                     gsems[b])

    def sdesc(s, b):
        return pltpu.make_async_copy(rows_v.at[b], acc_s.at[dst_ring.at[s]],
                                     ssems[b])

    for k in range(NSLOT - 1):
        isdesc(k, k).start()
        iddesc(k, k).start()
    ZR = 40
    for r in range(ZR):
        for i in range(D // L):
            rows_v[0, r, pl.ds(i * L, L)] = jnp.zeros((L,), jnp.float32)
    zdescs = [
        pltpu.make_async_copy(rows_v.at[0, pl.ds(0, ZR)],
                              acc_s.at[pl.ds(sid * RPT + k * ZR, ZR)],
                              ssems[0])
        for k in range(RPT // ZR)
    ]
    for zd in zdescs:
        zd.start()
    for zd in zdescs:
        zd.wait()
    plsc.subcore_barrier()
    for k in range(LA):
        isdesc(k, k).wait()
        gdesc(k, k).start()

    def chunk(j, r):
        b, s = r % NBUF, r % NSLOT
        gdesc(s, b).wait()
        iddesc(j, s).wait()
        sdesc(s, b).start(add=True)

        @pl.when(j >= 1)
        def _():
            sdesc((s + NSLOT - 1) % NSLOT, (b + NBUF - 1) % NBUF).wait()

        @pl.when(j + NSLOT - 1 < cpw)
        def _():
            isdesc(j + NSLOT - 1, (s + NSLOT - 1) % NSLOT).start()
            iddesc(j + NSLOT - 1, (s + NSLOT - 1) % NSLOT).start()

        @pl.when(j + LA < cpw)
        def _():
            isdesc(j + LA, (s + LA) % NSLOT).wait()
            gdesc((s + LA) % NSLOT, (b + LA) % NBUF).start()

    @pl.loop(0, cpw // PER)
    def _(g):
        for k in range(PER):
            chunk(g * PER + k, k)

    sdesc((PER - 1) % NSLOT, (PER - 1) % NBUF).wait()
    plsc.subcore_barrier()
    pltpu.sync_copy(acc_s.at[pl.ds(sid * RPT, RPT)],
                    out_hbm.at[cid, pl.ds(sid * RPT, RPT)])


_agg_call = pl.kernel(
    _agg_body,
    out_type=jax.ShapeDtypeStruct((NC, NPAD, D), jnp.float32),
    mesh=_mesh,
    scratch_types=[
        pltpu.VMEM((NSLOT, C), jnp.int32),
        pltpu.VMEM((NSLOT, C), jnp.int32),
        pltpu.VMEM((NBUF, C, D), jnp.float32),
    ] + [pltpu.SemaphoreType.DMA] * (2 * NBUF + 2 * NSLOT) + [
        pltpu.VMEM_SHARED((NPAD, D), jnp.float32),
    ],
)


def _dinv_block(cnt_blk):
    return lax.rsqrt(1.0 + cnt_blk[:, 0] + cnt_blk[:, 1])[:, None]


def _lin_body(x_ref, w_ref, cnt_ref, o_ref):
    o_ref[...] = (
        jnp.dot(x_ref[...], w_ref[...], preferred_element_type=jnp.float32)
        * _dinv_block(cnt_ref[...])
    )


def _mid_body(g_ref, y_ref, cnt_ref, b_ref, w_ref, o_ref):
    g = g_ref[...]
    dinv = _dinv_block(cnt_ref[...])
    h = jnp.maximum(dinv * (g[0] + g[1] + y_ref[...]) + b_ref[...], 0.0)
    o_ref[...] = (
        jnp.dot(h, w_ref[...], preferred_element_type=jnp.float32) * dinv
    )


def _fin_body(g_ref, y_ref, cnt_ref, b_ref, o_ref):
    g = g_ref[...]
    dinv = _dinv_block(cnt_ref[...])
    o_ref[...] = dinv * (g[0] + g[1] + y_ref[...]) + b_ref[...]


_row_spec = pl.BlockSpec((RB, D), lambda i: (i, 0))
_cnt_spec = pl.BlockSpec((RB, NC), lambda i: (i, 0))
_g_spec = pl.BlockSpec((NC, RB, D), lambda i: (0, i, 0))
_w_spec = pl.BlockSpec((D, D), lambda i: (0, 0))
_b_spec = pl.BlockSpec((1, D), lambda i: (0, 0))
_out_shape = jax.ShapeDtypeStruct((NPAD, D), jnp.float32)

_lin_call = pl.pallas_call(
    _lin_body, grid=(GRID,),
    in_specs=[_row_spec, _w_spec, _cnt_spec],
    out_specs=_row_spec, out_shape=_out_shape,
)

_mid_call = pl.pallas_call(
    _mid_body, grid=(GRID,),
    in_specs=[_g_spec, _row_spec, _cnt_spec, _b_spec, _w_spec],
    out_specs=_row_spec, out_shape=_out_shape,
)

_fin_call = pl.pallas_call(
    _fin_body, grid=(GRID,),
    in_specs=[_g_spec, _row_spec, _cnt_spec, _b_spec],
    out_specs=_row_spec, out_shape=jax.ShapeDtypeStruct((N, D), jnp.float32),
)


def kernel(x, edge_index, W1, b1, W2, b2):
    src = edge_index[0].astype(jnp.int32)
    dst = edge_index[1].astype(jnp.int32)
    apad = jnp.full((AEPAD - E,), N, jnp.int32)

    def _split3(flat):
        a0 = flat[:NS * CPW0 * C].reshape(NS, CPW0, C)
        a1 = flat[NS * CPW0 * C:].reshape(NS, CPW1, C)
        a1 = jnp.pad(a1, ((0, 0), (0, CPW0 - CPW1), (0, 0)),
                     constant_values=N)
        return jnp.concatenate([a0, a1], axis=0)

    srca = _split3(jnp.concatenate([src, apad]))
    dsta = _split3(jnp.concatenate([dst, apad]))
    b1r = b1.reshape(1, D)
    b2r = b2.reshape(1, D)

    cnt = _deg_call(dsta).T
    y1 = _lin_call(x, W1, cnt)
    g1 = _agg_call(y1, srca, dsta)
    y2 = _mid_call(g1, y1, cnt, b1r, W2)
    g2 = _agg_call(y2, srca, dsta)
    return _fin_call(g2, y2, cnt, b2r)


# speedup vs baseline: 3.5473x; 1.0210x over previous
import jax
import jax.numpy as jnp
from jax import lax
from jax.experimental import pallas as pl
from jax.experimental.pallas import tpu as pltpu
from jax.experimental.pallas import tpu_sc as plsc

N = 10000
D = 128
E = 320000

NC = 2
NS = 16
NW = NC * NS
L = 16

DC = 128
DCPW0 = 97
DCPW1 = 65
DTOT = NS * (DCPW0 + DCPW1)
DEPAD = DTOT * DC
C = 88
CPW0 = 216
CPW1 = 12
NBUF = 4
LA = NBUF - 1
NSLOT = 6
PER = 12
ATOT = NS * (CPW0 + CPW1)
AEPAD = ATOT * C
RPT = 640
NPAD = NS * RPT

RB = 400
GRID = N // RB

_mesh = plsc.VectorSubcoreMesh(
    core_axis_name="c", subcore_axis_name="s", num_cores=NC, num_subcores=NS
)


def _deg_body(dst_hbm, cnt_hbm, dst_v, ones_v, zeros_v, cnt_s):
    cid = lax.axis_index("c")
    sid = lax.axis_index("s")
    wid = cid * NS + sid
    dcpw = lax.select(cid == 0, DCPW0, DCPW1)
    for i in range(DC // L):
        ones_v[pl.ds(i * L, L)] = jnp.full((L,), 1.0, jnp.float32)
    for i in range(RPT // L):
        zeros_v[pl.ds(i * L, L)] = jnp.zeros((L,), jnp.float32)
    pltpu.sync_copy(dst_hbm.at[wid], dst_v)
    pltpu.sync_copy(zeros_v, cnt_s.at[pl.ds(sid * RPT, RPT)])
    plsc.subcore_barrier()

    @pl.loop(0, dcpw)
    def _(j):
        pltpu.sync_copy(ones_v, cnt_s.at[dst_v.at[j]], add=True)

    plsc.subcore_barrier()
    pltpu.sync_copy(cnt_s.at[pl.ds(sid * RPT, RPT)],
                    cnt_hbm.at[cid, pl.ds(sid * RPT, RPT)])


_deg_call = pl.kernel(
    _deg_body,
    out_type=jax.ShapeDtypeStruct((NC, NPAD), jnp.float32),
    mesh=_mesh,
    scratch_types=[
        pltpu.VMEM((DCPW0, DC), jnp.int32),
        pltpu.VMEM((DC,), jnp.float32),
        pltpu.VMEM((RPT,), jnp.float32),
        pltpu.VMEM_SHARED((NPAD,), jnp.float32),
    ],
)


def _agg_body(y_hbm, src_hbm, dst_hbm, out_hbm, src_ring, dst_ring, rows_v,
              *sems):
    gsems = sems[0:NBUF]
    ssems = sems[NBUF:2 * NBUF]
    isems = sems[2 * NBUF:2 * NBUF + NSLOT]
    idems = sems[2 * NBUF + NSLOT:2 * NBUF + 2 * NSLOT]
    acc_s = sems[-1]
    cid = lax.axis_index("c")
    sid = lax.axis_index("s")
    wid = cid * NS + sid
    cpw = lax.select(cid == 0, CPW0, CPW1)

    def isdesc(j, s):
        return pltpu.make_async_copy(src_hbm.at[wid, j], src_ring.at[s],
                                     isems[s])

    def iddesc(j, s):
        return pltpu.make_async_copy(dst_hbm.at[wid, j], dst_ring.at[s],
                                     idems[s])

    def gdesc(s, b):
        return pltpu.make_async_copy(y_hbm.at[src_ring.at[s]], rows_v.at[b],
                                     gsems[b])

    def sdesc(s, b):
        return pltpu.make_async_copy(rows_v.at[b], acc_s.at[dst_ring.at[s]],
                                     ssems[b])

    for k in range(NSLOT - 1):
        isdesc(k, k).start()
        iddesc(k, k).start()
    ZR = 40
    for r in range(ZR):
        for i in range(D // L):
            rows_v[0, r, pl.ds(i * L, L)] = jnp.zeros((L,), jnp.float32)
    zdescs = [
        pltpu.make_async_copy(rows_v.at[0, pl.ds(0, ZR)],
                              acc_s.at[pl.ds(sid * RPT + k * ZR, ZR)],
                              ssems[0])
        for k in range(RPT // ZR)
    ]
    for zd in zdescs:
        zd.start()
    for zd in zdescs:
        zd.wait()
    plsc.subcore_barrier()
    for k in range(LA):
        isdesc(k, k).wait()
        gdesc(k, k).start()

    def chunk(j, r):
        b, s = r % NBUF, r % NSLOT
        gdesc(s, b).wait()
        iddesc(j, s).wait()
        sdesc(s, b).start(add=True)

        @pl.when(j >= 1)
        def _():
            sdesc((s + NSLOT - 1) % NSLOT, (b + NBUF - 1) % NBUF).wait()

        @pl.when(j + NSLOT - 1 < cpw)
        def _():
            isdesc(j + NSLOT - 1, (s + NSLOT - 1) % NSLOT).start()
            iddesc(j + NSLOT - 1, (s + NSLOT - 1) % NSLOT).start()

        @pl.when(j + LA < cpw)
        def _():
            isdesc(j + LA, (s + LA) % NSLOT).wait()
            gdesc((s + LA) % NSLOT, (b + LA) % NBUF).start()

    @pl.loop(0, cpw // PER)
    def _(g):
        for k in range(PER):
            chunk(g * PER + k, k)

    sdesc((PER - 1) % NSLOT, (PER - 1) % NBUF).wait()
    plsc.subcore_barrier()
    pltpu.sync_copy(acc_s.at[pl.ds(sid * RPT, RPT)],
                    out_hbm.at[cid, pl.ds(sid * RPT, RPT)])


_agg_call = pl.kernel(
    _agg_body,
    out_type=jax.ShapeDtypeStruct((NC, NPAD, D), jnp.float32),
    mesh=_mesh,
    scratch_types=[
        pltpu.VMEM((NSLOT, C), jnp.int32),
        pltpu.VMEM((NSLOT, C), jnp.int32),
        pltpu.VMEM((NBUF, C, D), jnp.float32),
    ] + [pltpu.SemaphoreType.DMA] * (2 * NBUF + 2 * NSLOT) + [
        pltpu.VMEM_SHARED((NPAD, D), jnp.float32),
    ],
)


def _dinv_block(cnt_blk):
    return lax.rsqrt(1.0 + cnt_blk[:, 0] + cnt_blk[:, 1])[:, None]


def _lin_body(x_ref, w_ref, cnt_ref, o_ref):
    o_ref[...] = (
        jnp.dot(x_ref[...], w_ref[...], preferred_element_type=jnp.float32)
        * _dinv_block(cnt_ref[...])
    )


def _mid_body(g_ref, y_ref, cnt_ref, b_ref, w_ref, o_ref):
    g = g_ref[...]
    dinv = _dinv_block(cnt_ref[...])
    h = jnp.maximum(dinv * (g[0] + g[1] + y_ref[...]) + b_ref[...], 0.0)
    o_ref[...] = (
        jnp.dot(h, w_ref[...], preferred_element_type=jnp.float32) * dinv
    )


def _fin_body(g_ref, y_ref, cnt_ref, b_ref, o_ref):
    g = g_ref[...]
    dinv = _dinv_block(cnt_ref[...])
    o_ref[...] = dinv * (g[0] + g[1] + y_ref[...]) + b_ref[...]


_row_spec = pl.BlockSpec((RB, D), lambda i: (i, 0))
_cnt_spec = pl.BlockSpec((RB, NC), lambda i: (i, 0))
_g_spec = pl.BlockSpec((NC, RB, D), lambda i: (0, i, 0))
_w_spec = pl.BlockSpec((D, D), lambda i: (0, 0))
_b_spec = pl.BlockSpec((1, D), lambda i: (0, 0))
_out_shape = jax.ShapeDtypeStruct((NPAD, D), jnp.float32)

_lin_call = pl.pallas_call(
    _lin_body, grid=(GRID,),
    in_specs=[_row_spec, _w_spec, _cnt_spec],
    out_specs=_row_spec, out_shape=_out_shape,
)

_mid_call = pl.pallas_call(
    _mid_body, grid=(GRID,),
    in_specs=[_g_spec, _row_spec, _cnt_spec, _b_spec, _w_spec],
    out_specs=_row_spec, out_shape=_out_shape,
)

_fin_call = pl.pallas_call(
    _fin_body, grid=(GRID,),
    in_specs=[_g_spec, _row_spec, _cnt_spec, _b_spec],
    out_specs=_row_spec, out_shape=jax.ShapeDtypeStruct((N, D), jnp.float32),
)


def kernel(x, edge_index, W1, b1, W2, b2):
    src = edge_index[0].astype(jnp.int32)
    dst = edge_index[1].astype(jnp.int32)
    apad = jnp.full((AEPAD - E,), N, jnp.int32)
    dpad = jnp.full((DEPAD - E,), N, jnp.int32)

    def _split3(flat, n0, n1, c):
        a0 = flat[:NS * n0 * c].reshape(NS, n0, c)
        a1 = flat[NS * n0 * c:].reshape(NS, n1, c)
        a1 = jnp.pad(a1, ((0, 0), (0, n0 - n1), (0, 0)), constant_values=N)
        return jnp.concatenate([a0, a1], axis=0)

    srca = _split3(jnp.concatenate([src, apad]), CPW0, CPW1, C)
    dsta = _split3(jnp.concatenate([dst, apad]), CPW0, CPW1, C)
    dstd = _split3(jnp.concatenate([dst, dpad]), DCPW0, DCPW1, DC)
    b1r = b1.reshape(1, D)
    b2r = b2.reshape(1, D)

    cnt = _deg_call(dstd).T
    y1 = _lin_call(x, W1, cnt)
    g1 = _agg_call(y1, srca, dsta)
    y2 = _mid_call(g1, y1, cnt, b1r, W2)
    g2 = _agg_call(y2, srca, dsta)
    return _fin_call(g2, y2, cnt, b2r)
